# baseline (speedup 1.0000x reference)
import jax
import jax.numpy as jnp
from jax.experimental import pallas as pl

_BS = 512


def _add_body(x_ref, pe_ref, o_ref):
    i = pl.program_id(1)
    o_ref[...] = x_ref[...] + pe_ref[:, pl.ds(i * _BS, _BS), :]


def kernel(inputs, pos_embed):
    B, S, D = inputs.shape
    grid = (B, S // _BS)
    return pl.pallas_call(
        _add_body,
        grid=grid,
        in_specs=[
            pl.BlockSpec((1, _BS, D), lambda b, i: (b, i, 0)),
            pl.BlockSpec((1, S, D), lambda b, i: (0, 0, 0)),
        ],
        out_specs=pl.BlockSpec((1, _BS, D), lambda b, i: (b, i, 0)),
        out_shape=jax.ShapeDtypeStruct((B, S, D), inputs.dtype),
    )(inputs, pos_embed[None])


# speedup vs baseline: 1.0029x; 1.0029x over previous
import jax
import jax.numpy as jnp
from jax.experimental import pallas as pl

_BS = 512


def _make_body(S):
    nsb = S // _BS

    def _add_body(x_ref, pe_ref, o_ref):
        i = pl.program_id(0) % nsb
        o_ref[...] = x_ref[...] + pe_ref[pl.ds(i * _BS, _BS), :]

    return _add_body


def kernel(inputs, pos_embed):
    B, S, D = inputs.shape
    x2d = inputs.reshape(B * S, D)
    grid = (B * S // _BS,)
    out = pl.pallas_call(
        _make_body(S),
        grid=grid,
        in_specs=[
            pl.BlockSpec((_BS, D), lambda i: (i, 0)),
            pl.BlockSpec((S, D), lambda i: (0, 0)),
        ],
        out_specs=pl.BlockSpec((_BS, D), lambda i: (i, 0)),
        out_shape=jax.ShapeDtypeStruct((B * S, D), inputs.dtype),
    )(x2d, pos_embed)
    return out.reshape(B, S, D)


# speedup vs baseline: 1.1611x; 1.1578x over previous
import jax
import jax.numpy as jnp
from jax.experimental import pallas as pl


def _add_body(x_ref, pe_ref, o_ref):
    o_ref[...] = x_ref[...] + pe_ref[...]


def kernel(inputs, pos_embed):
    B, S, D = inputs.shape
    x2d = inputs.reshape(B * S, D)
    out = pl.pallas_call(
        _add_body,
        grid=(B,),
        in_specs=[
            pl.BlockSpec((S, D), lambda i: (i, 0)),
            pl.BlockSpec((S, D), lambda i: (0, 0)),
        ],
        out_specs=pl.BlockSpec((S, D), lambda i: (i, 0)),
        out_shape=jax.ShapeDtypeStruct((B * S, D), inputs.dtype),
    )(x2d, pos_embed)
    return out.reshape(B, S, D)
